# baseline (speedup 1.0000x reference)
import functools

import jax
import jax.numpy as jnp
from jax import lax
from jax.experimental import pallas as pl
from jax.experimental.pallas import tpu as pltpu
from jax.experimental.pallas import tpu_sc as plsc

N = 1000000
N_ZA = 32
N_ZS = 32
BATCH = 16384

_info = plsc.get_sparse_core_info()
_NC, _NS = _info.num_cores, _info.num_subcores
_NW = _NC * _NS
_BPW = BATCH // _NW


def _gather_body(idx_hbm, wa_hbm, ws_hbm, oa_hbm, os_hbm,
                 idx_v, ra_v, rs_v, sem_a, sem_s):
    wid = lax.axis_index("s") * _NC + lax.axis_index("c")
    base = wid * _BPW
    pltpu.sync_copy(idx_hbm.at[pl.ds(base, _BPW)], idx_v)
    ca = pltpu.async_copy(wa_hbm.at[idx_v], ra_v, sem_a)
    cs = pltpu.async_copy(ws_hbm.at[idx_v], rs_v, sem_s)
    ca.wait()
    pltpu.sync_copy(ra_v, oa_hbm.at[pl.ds(base, _BPW)])
    cs.wait()
    pltpu.sync_copy(rs_v, os_hbm.at[pl.ds(base, _BPW)])


@jax.jit
def kernel(idx, Wa, Ws):
    mesh = plsc.VectorSubcoreMesh(core_axis_name="c", subcore_axis_name="s")
    run = functools.partial(
        pl.kernel,
        mesh=mesh,
        out_type=(
            jax.ShapeDtypeStruct((BATCH, N_ZA), jnp.float32),
            jax.ShapeDtypeStruct((BATCH, N_ZS), jnp.float32),
        ),
        scratch_types=[
            pltpu.VMEM((_BPW,), jnp.int32),
            pltpu.VMEM((_BPW, N_ZA), jnp.float32),
            pltpu.VMEM((_BPW, N_ZS), jnp.float32),
            pltpu.SemaphoreType.DMA,
            pltpu.SemaphoreType.DMA,
        ],
        compiler_params=pltpu.CompilerParams(use_tc_tiling_on_sc=False),
    )(_gather_body)
    return run(idx, Wa, Ws)
